# baseline (speedup 1.0000x reference)
import functools
import math

import jax
import jax.numpy as jnp
from jax import lax
from jax.experimental import pallas as pl
from jax.experimental.pallas import tpu as pltpu
from jax.experimental.pallas import tpu_sc as plsc

VOCAB = 1000000
D = 32
SCALE = math.sqrt(32.0)
NC, NS = 2, 16
NW = NC * NS
B = 4096 * 200
BPW = B // NW
CH = 1600
NCH = BPW // CH
NBUF = 2
NROUNDS = NCH // NBUF

_mesh = plsc.VectorSubcoreMesh(core_axis_name="c", subcore_axis_name="s")


@functools.partial(
    pl.kernel,
    mesh=_mesh,
    out_type=jax.ShapeDtypeStruct((B, D), jnp.float32),
    scratch_types=[
        pltpu.VMEM((CH,), jnp.int32),
        pltpu.VMEM((CH,), jnp.int32),
        pltpu.VMEM((CH, D), jnp.float32),
        pltpu.VMEM((CH, D), jnp.float32),
        pltpu.SemaphoreType.DMA,
        pltpu.SemaphoreType.DMA,
        pltpu.SemaphoreType.DMA,
        pltpu.SemaphoreType.DMA,
    ],
    compiler_params=pltpu.CompilerParams(use_tc_tiling_on_sc=False),
)
def _gather(idx_hbm, table_hbm, out_hbm,
            idx0, idx1, rows0, rows1, sg0, sg1, sw0, sw1):
    wid = lax.axis_index("s") * NC + lax.axis_index("c")
    base = wid * BPW

    idx_b = (idx0, idx1)
    rows_b = (rows0, rows1)
    sg_b = (sg0, sg1)
    sw_b = (sw0, sw1)

    def stage_and_gather(b, c):
        off = base + c * CH
        pltpu.sync_copy(idx_hbm.at[pl.ds(off, CH)], idx_b[b])
        pltpu.async_copy(table_hbm.at[idx_b[b]], rows_b[b], sg_b[b])

    def start_write(b, c):
        off = base + c * CH
        return pltpu.async_copy(rows_b[b], out_hbm.at[pl.ds(off, CH)], sw_b[b])

    for b in range(NBUF):
        stage_and_gather(b, b)

    def round_body(r, carry):
        handles = []
        for b in range(NBUF):
            pltpu.make_async_copy(table_hbm.at[idx_b[b]], rows_b[b],
                                  sg_b[b]).wait()
            handles.append(start_write(b, r * NBUF + b))
        for b in range(NBUF):
            handles[b].wait()
            stage_and_gather(b, (r + 1) * NBUF + b)
        return carry

    lax.fori_loop(0, NROUNDS - 1, round_body, 0)

    handles = []
    for b in range(NBUF):
        pltpu.make_async_copy(table_hbm.at[idx_b[b]], rows_b[b],
                              sg_b[b]).wait()
        handles.append(start_write(b, (NROUNDS - 1) * NBUF + b))
    for b in range(NBUF):
        handles[b].wait()


def kernel(x, embedding):
    xf = x.reshape(-1).astype(jnp.int32)
    table_lin = lax.optimization_barrier(embedding.reshape(-1))
    table = table_lin.reshape(VOCAB, D)
    out = _gather(xf, table)
    return out.reshape(x.shape + (D,)) * jnp.float32(SCALE)


# speedup vs baseline: 1.1215x; 1.1215x over previous
import functools
import math

import jax
import jax.numpy as jnp
from jax import lax
from jax.experimental import pallas as pl
from jax.experimental.pallas import tpu as pltpu
from jax.experimental.pallas import tpu_sc as plsc

D = 32
SCALE = math.sqrt(32.0)
NC, NS, L = 2, 16, 16
NW = NC * NS
B = 4096 * 200
BPW = B // NW
CH = 1600
NCH = BPW // CH

_mesh = plsc.VectorSubcoreMesh(core_axis_name="c", subcore_axis_name="s")


@functools.partial(
    pl.kernel,
    mesh=_mesh,
    out_type=jax.ShapeDtypeStruct((B, D), jnp.float32),
    scratch_types=[
        pltpu.VMEM((CH,), jnp.int32),
        pltpu.VMEM((CH, D), jnp.float32),
        pltpu.SemaphoreType.DMA,
    ],
    compiler_params=pltpu.CompilerParams(use_tc_tiling_on_sc=False),
)
def _gather_scale(idx_hbm, table_hbm, out_hbm, idx_v, rows_v, sem):
    wid = lax.axis_index("s") * NC + lax.axis_index("c")
    base = wid * BPW

    def chunk_body(c, carry):
        off = base + c * CH
        pltpu.sync_copy(idx_hbm.at[pl.ds(off, CH)], idx_v)
        pltpu.async_copy(table_hbm.at[idx_v], rows_v, sem).wait()

        def row_body(i, carry2):
            for h in range(D // L):
                v = rows_v[i, pl.ds(h * L, L)]
                rows_v[i, pl.ds(h * L, L)] = v * SCALE
            return carry2

        lax.fori_loop(0, CH, row_body, 0)
        pltpu.sync_copy(rows_v, out_hbm.at[pl.ds(off, CH)])
        return carry

    lax.fori_loop(0, NCH, chunk_body, 0)


def kernel(x, embedding):
    xf = x.reshape(-1).astype(jnp.int32)
    out = _gather_scale(xf, embedding)
    return out.reshape(x.shape + (D,))
